# Initial kernel scaffold; baseline (speedup 1.0000x reference)
import jax
import jax.numpy as jnp
from jax.experimental import pallas as pl


def kernel(x, pos_emb):
    raise NotImplementedError("write your pallas kernel here")


# speedup vs baseline: 2.8505x; 2.8505x over previous
import jax
import jax.numpy as jnp
from jax.experimental import pallas as pl

NUM_POSITIONS = 8192
HIDDEN = 1024
BL = 512


def _add_kernel(x_ref, emb_ref, o_ref):
    o_ref[...] = x_ref[...] + emb_ref[...]


def kernel(x, pos_emb):
    b, l, h = x.shape
    num_l_blocks = l // BL

    grid = (num_l_blocks, b)

    return pl.pallas_call(
        _add_kernel,
        grid=grid,
        in_specs=[
            pl.BlockSpec((1, BL, h), lambda i, j: (j, i, 0)),
            pl.BlockSpec((BL, h), lambda i, j: (i, 0)),
        ],
        out_specs=pl.BlockSpec((1, BL, h), lambda i, j: (j, i, 0)),
        out_shape=jax.ShapeDtypeStruct((b, l, h), x.dtype),
    )(x, pos_emb)
